# Initial kernel scaffold; baseline (speedup 1.0000x reference)
import jax
import jax.numpy as jnp
from jax.experimental import pallas as pl


def kernel(indices, values, shape, b):
    raise NotImplementedError("write your pallas kernel here")


# speedup vs baseline: 2.1574x; 2.1574x over previous
import functools

import jax
import jax.numpy as jnp
from jax import lax
from jax.experimental import pallas as pl
from jax.experimental.pallas import tpu as pltpu
from jax.experimental.pallas import tpu_sc as plsc

B, N, D = 8, 4096, 1024
CHUNK = 8
NCHUNK = D // CHUNK
NC, NS = 2, 16
NW = NC * NS
TASKS_PER_W = B * NCHUNK // NW
PIECE = 14336
NP = 12
NNZPAD = PIECE * NP


def _body(rows_hbm, cols_hbm, vals_hbm, bt_hbm, out_hbm,
          bloc, accum, ridx, cidx, vbuf):
    wid = lax.axis_index("c") * NS + lax.axis_index("s")
    bi = wid // (NCHUNK // TASKS_PER_W)
    cbase = (wid % (NCHUNK // TASKS_PER_W)) * TASKS_PER_W

    zeros16 = jnp.zeros((16,), jnp.float32)

    def task(t, _):
        c = cbase + t
        pltpu.sync_copy(bt_hbm.at[bi, c], bloc)

        def zero(j, _):
            accum[pl.ds(j * 16, 16)] = zeros16
            return 0
        lax.fori_loop(0, N * CHUNK // 16, zero, 0)

        def piece(p, _):
            off = p * PIECE
            pltpu.sync_copy(rows_hbm.at[bi, pl.ds(off, PIECE)], ridx)
            pltpu.sync_copy(cols_hbm.at[bi, pl.ds(off, PIECE)], cidx)
            pltpu.sync_copy(vals_hbm.at[bi, pl.ds(off, PIECE)], vbuf)

            def step(i, _):
                r = ridx[pl.ds(i * 16, 16)]
                cc = cidx[pl.ds(i * 16, 16)]
                v = vbuf[pl.ds(i * 16, 16)]
                r8 = r << 3
                c8 = cc << 3
                gs = [plsc.load_gather(bloc, [c8 + d]) for d in range(CHUNK)]
                ps = [g * v for g in gs]
                for d in range(CHUNK):
                    plsc.addupdate_scatter(accum, [r8 + d], ps[d])
                return 0
            lax.fori_loop(0, PIECE // 16, step, 0)
            return 0
        lax.fori_loop(0, NP, piece, 0)

        pltpu.sync_copy(accum, out_hbm.at[bi, c])
        return 0

    lax.fori_loop(0, TASKS_PER_W, task, 0)


@jax.jit
def _spmm(rows, cols, vals, bt):
    mesh = plsc.VectorSubcoreMesh(core_axis_name="c", subcore_axis_name="s",
                                  num_cores=NC, num_subcores=NS)
    f = pl.kernel(
        _body,
        out_type=jax.ShapeDtypeStruct((B, NCHUNK, N * CHUNK), jnp.float32),
        mesh=mesh,
        scratch_types=[
            pltpu.VMEM((N * CHUNK,), jnp.float32),
            pltpu.VMEM((N * CHUNK,), jnp.float32),
            pltpu.VMEM((PIECE,), jnp.int32),
            pltpu.VMEM((PIECE,), jnp.int32),
            pltpu.VMEM((PIECE,), jnp.float32),
        ],
        compiler_params=pltpu.CompilerParams(needs_layout_passes=False),
    )
    return f(rows, cols, vals, bt)


def kernel(indices, values, shape, b):
    nnz = indices.shape[-1]
    rows = indices[:, 0, :].astype(jnp.int32)
    cols = indices[:, 1, :].astype(jnp.int32)
    vals = values.astype(jnp.float32)
    pad = NNZPAD - nnz
    rows = jnp.pad(rows, ((0, 0), (0, pad)))
    cols = jnp.pad(cols, ((0, 0), (0, pad)))
    vals = jnp.pad(vals, ((0, 0), (0, pad)))
    bt = b.reshape(B, N, NCHUNK, CHUNK).transpose(0, 2, 1, 3) \
         .reshape(B, NCHUNK, N * CHUNK)
    out = _spmm(rows, cols, vals, bt)
    return out.reshape(B, NCHUNK, N, CHUNK).transpose(0, 2, 1, 3) \
              .reshape(B, N, D)


# speedup vs baseline: 2.3493x; 1.0889x over previous
import jax
import jax.numpy as jnp
from jax import lax
from jax.experimental import pallas as pl
from jax.experimental.pallas import tpu as pltpu
from jax.experimental.pallas import tpu_sc as plsc

B, N, D = 8, 4096, 1024
CHUNK = 8
NCHUNK = D // CHUNK
NC, NS = 2, 16
NW = NC * NS
TASKS_PER_W = B * NCHUNK // NW
PIECE = 7168
NP = 24
NNZPAD = PIECE * NP


def _body(rows_hbm, cols_hbm, vals_hbm, bt_hbm, out_hbm,
          bloc, accum, ridx, cidx, vbuf, sem0, sem1):
    wid = lax.axis_index("c") * NS + lax.axis_index("s")
    bi = wid // (NCHUNK // TASKS_PER_W)
    cbase = (wid % (NCHUNK // TASKS_PER_W)) * TASKS_PER_W

    zeros16 = jnp.zeros((16,), jnp.float32)
    sems = (sem0, sem1)

    def start_piece(q, bsel):
        off = q * PIECE
        dst = pl.ds(bsel * PIECE, PIECE)
        pltpu.async_copy(rows_hbm.at[bi, pl.ds(off, PIECE)], ridx.at[dst],
                         sems[bsel])
        pltpu.async_copy(cols_hbm.at[bi, pl.ds(off, PIECE)], cidx.at[dst],
                         sems[bsel])
        pltpu.async_copy(vals_hbm.at[bi, pl.ds(off, PIECE)], vbuf.at[dst],
                         sems[bsel])

    def wait_piece(q, bsel):
        off = q * PIECE
        dst = pl.ds(bsel * PIECE, PIECE)
        pltpu.make_async_copy(rows_hbm.at[bi, pl.ds(off, PIECE)], ridx.at[dst],
                              sems[bsel]).wait()
        pltpu.make_async_copy(cols_hbm.at[bi, pl.ds(off, PIECE)], cidx.at[dst],
                              sems[bsel]).wait()
        pltpu.make_async_copy(vals_hbm.at[bi, pl.ds(off, PIECE)], vbuf.at[dst],
                              sems[bsel]).wait()

    def task(t, _):
        c = cbase + t
        pltpu.sync_copy(bt_hbm.at[bi, c], bloc)

        def zero(j, _):
            for u in range(4):
                accum[pl.ds(j * 64 + u * 16, 16)] = zeros16
            return 0
        lax.fori_loop(0, N * CHUNK // 64, zero, 0)

        start_piece(0, 0)

        def outer(i, _):
            for bsel in range(2):
                q = 2 * i + bsel

                @pl.when(q + 1 < NP)
                def _():
                    start_piece(q + 1, 1 - bsel)

                wait_piece(q, bsel)

                def step(k, _):
                    base = bsel * PIECE + k * 32
                    for u in range(2):
                        o = base + u * 16
                        r = ridx[pl.ds(o, 16)]
                        cc = cidx[pl.ds(o, 16)]
                        v = vbuf[pl.ds(o, 16)]
                        r8 = r << 3
                        c8 = cc << 3
                        gs = [plsc.load_gather(bloc, [c8 + d])
                              for d in range(CHUNK)]
                        ps = [g * v for g in gs]
                        for d in range(CHUNK):
                            plsc.addupdate_scatter(accum, [r8 + d], ps[d])
                    return 0
                lax.fori_loop(0, PIECE // 32, step, 0)
            return 0
        lax.fori_loop(0, NP // 2, outer, 0)

        pltpu.sync_copy(accum, out_hbm.at[bi, c])
        return 0

    lax.fori_loop(0, TASKS_PER_W, task, 0)


@jax.jit
def _spmm(rows, cols, vals, bt):
    mesh = plsc.VectorSubcoreMesh(core_axis_name="c", subcore_axis_name="s",
                                  num_cores=NC, num_subcores=NS)
    f = pl.kernel(
        _body,
        out_type=jax.ShapeDtypeStruct((B, NCHUNK, N * CHUNK), jnp.float32),
        mesh=mesh,
        scratch_types=[
            pltpu.VMEM((N * CHUNK,), jnp.float32),
            pltpu.VMEM((N * CHUNK,), jnp.float32),
            pltpu.VMEM((2 * PIECE,), jnp.int32),
            pltpu.VMEM((2 * PIECE,), jnp.int32),
            pltpu.VMEM((2 * PIECE,), jnp.float32),
            pltpu.SemaphoreType.DMA,
            pltpu.SemaphoreType.DMA,
        ],
        compiler_params=pltpu.CompilerParams(needs_layout_passes=False),
    )
    return f(rows, cols, vals, bt)


def kernel(indices, values, shape, b):
    nnz = indices.shape[-1]
    rows = indices[:, 0, :].astype(jnp.int32)
    cols = indices[:, 1, :].astype(jnp.int32)
    vals = values.astype(jnp.float32)
    pad = NNZPAD - nnz
    rows = jnp.pad(rows, ((0, 0), (0, pad)))
    cols = jnp.pad(cols, ((0, 0), (0, pad)))
    vals = jnp.pad(vals, ((0, 0), (0, pad)))
    bt = b.reshape(B, N, NCHUNK, CHUNK).transpose(0, 2, 1, 3) \
         .reshape(B, NCHUNK, N * CHUNK)
    out = _spmm(rows, cols, vals, bt)
    return out.reshape(B, NCHUNK, N, CHUNK).transpose(0, 2, 1, 3) \
              .reshape(B, N, D)


# speedup vs baseline: 4.0927x; 1.7421x over previous
import jax
import jax.numpy as jnp
from jax import lax
from jax.experimental import pallas as pl
from jax.experimental.pallas import tpu as pltpu
from jax.experimental.pallas import tpu_sc as plsc

B, N, D = 8, 4096, 1024
CHUNK = 8
NCHUNK = D // CHUNK
NC, NS = 2, 16
NW = NC * NS
TASKS_PER_W = B * NCHUNK // NW
PIECE = 7168
NP = 24
NNZPAD = PIECE * NP


def _body(rows_hbm, cols_hbm, vals_hbm, bt_hbm, out_hbm,
          bloc, accum, ridx, cidx, vbuf, sem0, sem1):
    wid = lax.axis_index("c") * NS + lax.axis_index("s")
    bi = wid // (NCHUNK // TASKS_PER_W)
    cbase = (wid % (NCHUNK // TASKS_PER_W)) * TASKS_PER_W

    zeros16 = jnp.zeros((16,), jnp.float32)
    sems = (sem0, sem1)

    def start_piece(q, bsel):
        off = q * PIECE
        dst = pl.ds(bsel * PIECE, PIECE)
        pltpu.async_copy(rows_hbm.at[bi, pl.ds(off, PIECE)], ridx.at[dst],
                         sems[bsel])
        pltpu.async_copy(cols_hbm.at[bi, pl.ds(off, PIECE)], cidx.at[dst],
                         sems[bsel])
        pltpu.async_copy(vals_hbm.at[bi, pl.ds(off, PIECE)], vbuf.at[dst],
                         sems[bsel])

    def wait_piece(q, bsel):
        off = q * PIECE
        dst = pl.ds(bsel * PIECE, PIECE)
        pltpu.make_async_copy(rows_hbm.at[bi, pl.ds(off, PIECE)], ridx.at[dst],
                              sems[bsel]).wait()
        pltpu.make_async_copy(cols_hbm.at[bi, pl.ds(off, PIECE)], cidx.at[dst],
                              sems[bsel]).wait()
        pltpu.make_async_copy(vals_hbm.at[bi, pl.ds(off, PIECE)], vbuf.at[dst],
                              sems[bsel]).wait()

    def task(t, _):
        c = cbase + t
        pltpu.sync_copy(bt_hbm.at[bi, c], bloc)

        def zero(j, _):
            for u in range(4):
                accum[pl.ds(j * 64 + u * 16, 16)] = zeros16
            return 0
        lax.fori_loop(0, N * CHUNK // 64, zero, 0)

        start_piece(0, 0)

        def outer(i, _):
            for bsel in range(2):
                q = 2 * i + bsel

                @pl.when(q + 1 < NP)
                def _():
                    start_piece(q + 1, 1 - bsel)

                wait_piece(q, bsel)

                base = bsel * PIECE
                r0 = ridx[pl.ds(base, 16)]
                c0 = cidx[pl.ds(base, 16)]
                v0 = vbuf[pl.ds(base, 16)]

                def step(k, carry):
                    rr, cc, v = carry
                    o = base + k * 16 + 16
                    rn = ridx[pl.ds(o, 16)]
                    cn = cidx[pl.ds(o, 16)]
                    vn = vbuf[pl.ds(o, 16)]
                    gs = [plsc.load_gather(bloc, [cc | (d * N)])
                          for d in range(CHUNK)]
                    ps = [g * v for g in gs]
                    for d in range(CHUNK):
                        plsc.addupdate_scatter(accum, [rr | (d * N)], ps[d])
                    return (rn, cn, vn)
                lax.fori_loop(0, PIECE // 16, step, (r0, c0, v0))
            return 0
        lax.fori_loop(0, NP // 2, outer, 0)

        pltpu.sync_copy(accum, out_hbm.at[bi, c])
        return 0

    lax.fori_loop(0, TASKS_PER_W, task, 0)


@jax.jit
def _spmm(rows, cols, vals, bt):
    mesh = plsc.VectorSubcoreMesh(core_axis_name="c", subcore_axis_name="s",
                                  num_cores=NC, num_subcores=NS)
    f = pl.kernel(
        _body,
        out_type=jax.ShapeDtypeStruct((B, NCHUNK, N * CHUNK), jnp.float32),
        mesh=mesh,
        scratch_types=[
            pltpu.VMEM((N * CHUNK,), jnp.float32),
            pltpu.VMEM((N * CHUNK,), jnp.float32),
            pltpu.VMEM((2 * PIECE + 16,), jnp.int32),
            pltpu.VMEM((2 * PIECE + 16,), jnp.int32),
            pltpu.VMEM((2 * PIECE + 16,), jnp.float32),
            pltpu.SemaphoreType.DMA,
            pltpu.SemaphoreType.DMA,
        ],
        compiler_params=pltpu.CompilerParams(needs_layout_passes=False),
    )
    return f(rows, cols, vals, bt)


def kernel(indices, values, shape, b):
    nnz = indices.shape[-1]
    rows = indices[:, 0, :].astype(jnp.int32)
    cols = indices[:, 1, :].astype(jnp.int32)
    vals = values.astype(jnp.float32)
    pad = NNZPAD - nnz
    rows = jnp.pad(rows, ((0, 0), (0, pad)))
    cols = jnp.pad(cols, ((0, 0), (0, pad)))
    vals = jnp.pad(vals, ((0, 0), (0, pad)))
    bt = b.reshape(B, N, NCHUNK, CHUNK).transpose(0, 2, 3, 1) \
         .reshape(B, NCHUNK, CHUNK * N)
    out = _spmm(rows, cols, vals, bt)
    return out.reshape(B, NCHUNK, CHUNK, N).transpose(0, 3, 1, 2) \
              .reshape(B, N, D)


# speedup vs baseline: 4.1365x; 1.0107x over previous
import jax
import jax.numpy as jnp
from jax import lax
from jax.experimental import pallas as pl
from jax.experimental.pallas import tpu as pltpu
from jax.experimental.pallas import tpu_sc as plsc

B, N, D = 8, 4096, 1024
CHUNK = 8
NCHUNK = D // CHUNK
NC, NS = 2, 16
NW = NC * NS
TASKS_PER_W = B * NCHUNK // NW
PIECE = 7168
NP = 24
NNZPAD = PIECE * NP


def _body(rows_hbm, cols_hbm, vals_hbm, bt_hbm, out_hbm,
          bloc, accum, ridx, cidx, vbuf, sem0, sem1, semw):
    wid = lax.axis_index("c") * NS + lax.axis_index("s")
    bi = wid // (NCHUNK // TASKS_PER_W)
    cbase = (wid % (NCHUNK // TASKS_PER_W)) * TASKS_PER_W

    zeros16 = jnp.zeros((16,), jnp.float32)
    sems = (sem0, sem1)

    def start_piece(q, bsel):
        off = q * PIECE
        dst = pl.ds(bsel * PIECE, PIECE)
        pltpu.async_copy(rows_hbm.at[bi, pl.ds(off, PIECE)], ridx.at[dst],
                         sems[bsel])
        pltpu.async_copy(cols_hbm.at[bi, pl.ds(off, PIECE)], cidx.at[dst],
                         sems[bsel])
        pltpu.async_copy(vals_hbm.at[bi, pl.ds(off, PIECE)], vbuf.at[dst],
                         sems[bsel])

    def wait_piece(q, bsel):
        off = q * PIECE
        dst = pl.ds(bsel * PIECE, PIECE)
        pltpu.make_async_copy(rows_hbm.at[bi, pl.ds(off, PIECE)], ridx.at[dst],
                              sems[bsel]).wait()
        pltpu.make_async_copy(cols_hbm.at[bi, pl.ds(off, PIECE)], cidx.at[dst],
                              sems[bsel]).wait()
        pltpu.make_async_copy(vals_hbm.at[bi, pl.ds(off, PIECE)], vbuf.at[dst],
                              sems[bsel]).wait()

    def task(t, _):
        c = cbase + t
        pltpu.sync_copy(bt_hbm.at[bi, c], bloc)

        start_piece(0, 0)

        @pl.when(t > 0)
        def _():
            pltpu.make_async_copy(accum, out_hbm.at[bi, c], semw).wait()

        def zero(j, _):
            for u in range(4):
                accum[pl.ds(j * 64 + u * 16, 16)] = zeros16
            return 0
        lax.fori_loop(0, N * CHUNK // 64, zero, 0)

        def outer(i, _):
            for bsel in range(2):
                q = 2 * i + bsel

                @pl.when(q + 1 < NP)
                def _():
                    start_piece(q + 1, 1 - bsel)

                wait_piece(q, bsel)

                base = bsel * PIECE
                r0 = ridx[pl.ds(base, 16)]
                c0 = cidx[pl.ds(base, 16)]
                v0 = vbuf[pl.ds(base, 16)]

                def step(k, carry):
                    rr, cc, v = carry
                    o = base + k * 16 + 16
                    rn = ridx[pl.ds(o, 16)]
                    cn = cidx[pl.ds(o, 16)]
                    vn = vbuf[pl.ds(o, 16)]
                    gs = [plsc.load_gather(bloc, [cc | (d * N)])
                          for d in range(CHUNK)]
                    ps = [g * v for g in gs]
                    for d in range(CHUNK):
                        plsc.addupdate_scatter(accum, [rr | (d * N)], ps[d])
                    return (rn, cn, vn)
                lax.fori_loop(0, PIECE // 16, step, (r0, c0, v0))
            return 0
        lax.fori_loop(0, NP // 2, outer, 0)

        pltpu.async_copy(accum, out_hbm.at[bi, c], semw)
        return 0

    lax.fori_loop(0, TASKS_PER_W, task, 0)
    pltpu.make_async_copy(accum, out_hbm.at[bi, cbase], semw).wait()


@jax.jit
def _spmm(rows, cols, vals, bt):
    mesh = plsc.VectorSubcoreMesh(core_axis_name="c", subcore_axis_name="s",
                                  num_cores=NC, num_subcores=NS)
    f = pl.kernel(
        _body,
        out_type=jax.ShapeDtypeStruct((B, NCHUNK, N * CHUNK), jnp.float32),
        mesh=mesh,
        scratch_types=[
            pltpu.VMEM((N * CHUNK,), jnp.float32),
            pltpu.VMEM((N * CHUNK,), jnp.float32),
            pltpu.VMEM((2 * PIECE + 16,), jnp.int32),
            pltpu.VMEM((2 * PIECE + 16,), jnp.int32),
            pltpu.VMEM((2 * PIECE + 16,), jnp.float32),
            pltpu.SemaphoreType.DMA,
            pltpu.SemaphoreType.DMA,
            pltpu.SemaphoreType.DMA,
        ],
        compiler_params=pltpu.CompilerParams(needs_layout_passes=False),
    )
    return f(rows, cols, vals, bt)


def kernel(indices, values, shape, b):
    nnz = indices.shape[-1]
    rows = indices[:, 0, :].astype(jnp.int32)
    cols = indices[:, 1, :].astype(jnp.int32)
    vals = values.astype(jnp.float32)
    pad = NNZPAD - nnz
    rows = jnp.pad(rows, ((0, 0), (0, pad)))
    cols = jnp.pad(cols, ((0, 0), (0, pad)))
    vals = jnp.pad(vals, ((0, 0), (0, pad)))
    bt = b.reshape(B, N, NCHUNK, CHUNK).transpose(0, 2, 3, 1) \
         .reshape(B, NCHUNK, CHUNK * N)
    out = _spmm(rows, cols, vals, bt)
    return out.reshape(B, NCHUNK, CHUNK, N).transpose(0, 3, 1, 2) \
              .reshape(B, N, D)


# speedup vs baseline: 4.6691x; 1.1288x over previous
import jax
import jax.numpy as jnp
from jax import lax
from jax.experimental import pallas as pl
from jax.experimental.pallas import tpu as pltpu
from jax.experimental.pallas import tpu_sc as plsc

B, N, D = 8, 4096, 1024
CHUNK = 8
NCHUNK = D // CHUNK
NC, NS = 2, 16
NW = NC * NS
TASKS_PER_W = B * NCHUNK // NW
PIECE = 7168
NP = 24
NNZPAD = PIECE * NP


def _body(rows_hbm, cols_hbm, vals_hbm, bt_hbm, out_hbm,
          bloc, accum, ridx, cidx, vbuf, sem0, sem1, semw):
    wid = lax.axis_index("c") * NS + lax.axis_index("s")
    bi = wid // (NCHUNK // TASKS_PER_W)
    cbase = (wid % (NCHUNK // TASKS_PER_W)) * TASKS_PER_W

    zeros16 = jnp.zeros((16,), jnp.float32)
    sems = (sem0, sem1)

    def start_piece(q, bsel):
        off = q * PIECE
        dst = pl.ds(bsel * PIECE, PIECE)
        pltpu.async_copy(rows_hbm.at[bi, pl.ds(off, PIECE)], ridx.at[dst],
                         sems[bsel])
        pltpu.async_copy(cols_hbm.at[bi, pl.ds(off, PIECE)], cidx.at[dst],
                         sems[bsel])
        pltpu.async_copy(vals_hbm.at[bi, pl.ds(off, PIECE)], vbuf.at[dst],
                         sems[bsel])

    def wait_piece(q, bsel):
        off = q * PIECE
        dst = pl.ds(bsel * PIECE, PIECE)
        pltpu.make_async_copy(rows_hbm.at[bi, pl.ds(off, PIECE)], ridx.at[dst],
                              sems[bsel]).wait()
        pltpu.make_async_copy(cols_hbm.at[bi, pl.ds(off, PIECE)], cidx.at[dst],
                              sems[bsel]).wait()
        pltpu.make_async_copy(vals_hbm.at[bi, pl.ds(off, PIECE)], vbuf.at[dst],
                              sems[bsel]).wait()

    def task(t, _):
        c = cbase + t
        pltpu.sync_copy(bt_hbm.at[bi, c], bloc)

        start_piece(0, 0)

        @pl.when(t > 0)
        def _():
            pltpu.make_async_copy(accum, out_hbm.at[bi, c], semw).wait()

        def zero(j, _):
            for u in range(4):
                accum[pl.ds(j * 64 + u * 16, 16)] = zeros16
            return 0
        lax.fori_loop(0, N * CHUNK // 64, zero, 0)

        def outer(i, _):
            for bsel in range(2):
                q = 2 * i + bsel

                @pl.when(q + 1 < NP)
                def _():
                    start_piece(q + 1, 1 - bsel)

                wait_piece(q, bsel)

                base = bsel * PIECE
                r0 = ridx[pl.ds(base, 16)]
                c0 = cidx[pl.ds(base, 16)]
                v0 = vbuf[pl.ds(base, 16)]

                def step(k, carry):
                    rr, cc, v = carry
                    o = base + k * 16 + 16
                    rn = ridx[pl.ds(o, 16)]
                    cn = cidx[pl.ds(o, 16)]
                    vn = vbuf[pl.ds(o, 16)]
                    gs = [plsc.load_gather(bloc, [cc | (d * N)])
                          for d in range(CHUNK)]
                    ps = [g * v for g in gs]
                    for d in range(CHUNK):
                        plsc.addupdate_scatter(accum, [rr | (d * N)], ps[d])
                    return (rn, cn, vn)
                lax.fori_loop(0, PIECE // 16, step, (r0, c0, v0))
            return 0
        lax.fori_loop(0, NP // 2, outer, 0)

        pltpu.async_copy(accum, out_hbm.at[bi, c], semw)
        return 0

    lax.fori_loop(0, TASKS_PER_W, task, 0)
    pltpu.make_async_copy(accum, out_hbm.at[bi, cbase], semw).wait()


@jax.jit
def _spmm(rows, cols, vals, bt):
    mesh = plsc.VectorSubcoreMesh(core_axis_name="c", subcore_axis_name="s",
                                  num_cores=NC, num_subcores=NS)
    f = pl.kernel(
        _body,
        out_type=jax.ShapeDtypeStruct((B, NCHUNK, N * CHUNK), jnp.float32),
        mesh=mesh,
        scratch_types=[
            pltpu.VMEM((N * CHUNK,), jnp.float32),
            pltpu.VMEM((N * CHUNK,), jnp.float32),
            pltpu.VMEM((2 * PIECE + 16,), jnp.int32),
            pltpu.VMEM((2 * PIECE + 16,), jnp.int32),
            pltpu.VMEM((2 * PIECE + 16,), jnp.float32),
            pltpu.SemaphoreType.DMA,
            pltpu.SemaphoreType.DMA,
            pltpu.SemaphoreType.DMA,
        ],
        compiler_params=pltpu.CompilerParams(needs_layout_passes=False),
    )
    return f(rows, cols, vals, bt)


def kernel(indices, values, shape, b):
    nnz = indices.shape[-1]
    rows = indices[:, 0, :].astype(jnp.int32)
    cols = indices[:, 1, :].astype(jnp.int32)
    vals = values.astype(jnp.float32)
    pad = NNZPAD - nnz
    rows = jnp.pad(rows, ((0, 0), (0, pad)))
    cols = jnp.pad(cols, ((0, 0), (0, pad)))
    vals = jnp.pad(vals, ((0, 0), (0, pad)))
    bank = jnp.where(jnp.arange(NNZPAD) < nnz, rows & 15, 16)
    order = jnp.argsort(bank, axis=-1, stable=True)
    perm = order.reshape(B, 16, NNZPAD // 16).transpose(0, 2, 1) \
                .reshape(B, NNZPAD)
    rows = jnp.take_along_axis(rows, perm, axis=-1)
    cols = jnp.take_along_axis(cols, perm, axis=-1)
    vals = jnp.take_along_axis(vals, perm, axis=-1)
    bt = b.reshape(B, N, NCHUNK, CHUNK).transpose(0, 2, 3, 1) \
         .reshape(B, NCHUNK, CHUNK * N)
    out = _spmm(rows, cols, vals, bt)
    return out.reshape(B, NCHUNK, CHUNK, N).transpose(0, 3, 1, 2) \
              .reshape(B, N, D)


# speedup vs baseline: 5.0842x; 1.0889x over previous
import jax
import jax.numpy as jnp
from jax import lax
from jax.experimental import pallas as pl
from jax.experimental.pallas import tpu as pltpu
from jax.experimental.pallas import tpu_sc as plsc

B, N, D = 8, 4096, 1024
CHUNK = 8
NCHUNK = D // CHUNK
NC, NS = 2, 16
NW = NC * NS
TASKS_PER_W = B * NCHUNK // NW
PIECE = 7168
NP = 24
NNZPAD = PIECE * NP


def _body(rows_hbm, cols_hbm, vals_hbm, bt_hbm, out_hbm,
          bloc, accum, ridx, cidx, vbuf, sem0, sem1, semw):
    wid = lax.axis_index("c") * NS + lax.axis_index("s")
    bi = wid // (NCHUNK // TASKS_PER_W)
    cbase = (wid % (NCHUNK // TASKS_PER_W)) * TASKS_PER_W

    zeros16 = jnp.zeros((16,), jnp.float32)
    sems = (sem0, sem1)

    def start_piece(q, bsel):
        off = q * PIECE
        dst = pl.ds(bsel * PIECE, PIECE)
        pltpu.async_copy(rows_hbm.at[bi, pl.ds(off, PIECE)], ridx.at[dst],
                         sems[bsel])
        pltpu.async_copy(cols_hbm.at[bi, pl.ds(off, PIECE)], cidx.at[dst],
                         sems[bsel])
        pltpu.async_copy(vals_hbm.at[bi, pl.ds(off, PIECE)], vbuf.at[dst],
                         sems[bsel])

    def wait_piece(q, bsel):
        off = q * PIECE
        dst = pl.ds(bsel * PIECE, PIECE)
        pltpu.make_async_copy(rows_hbm.at[bi, pl.ds(off, PIECE)], ridx.at[dst],
                              sems[bsel]).wait()
        pltpu.make_async_copy(cols_hbm.at[bi, pl.ds(off, PIECE)], cidx.at[dst],
                              sems[bsel]).wait()
        pltpu.make_async_copy(vals_hbm.at[bi, pl.ds(off, PIECE)], vbuf.at[dst],
                              sems[bsel]).wait()

    def task(t, _):
        c = cbase + t
        pltpu.sync_copy(bt_hbm.at[bi, c], bloc)

        start_piece(0, 0)

        @pl.when(t > 0)
        def _():
            pltpu.make_async_copy(accum, out_hbm.at[bi, c], semw).wait()

        def zero(j, _):
            for u in range(4):
                accum[pl.ds(j * 64 + u * 16, 16)] = zeros16
            return 0
        lax.fori_loop(0, N * CHUNK // 64, zero, 0)

        def outer(i, _):
            for bsel in range(2):
                q = 2 * i + bsel

                @pl.when(q + 1 < NP)
                def _():
                    start_piece(q + 1, 1 - bsel)

                wait_piece(q, bsel)

                base = bsel * PIECE
                r0 = ridx[pl.ds(base, 16)]
                c0 = cidx[pl.ds(base, 16)]
                v0 = vbuf[pl.ds(base, 16)]

                def step(k, carry):
                    rr, cc, v = carry
                    o = base + k * 16 + 16
                    rn = ridx[pl.ds(o, 16)]
                    cn = cidx[pl.ds(o, 16)]
                    vn = vbuf[pl.ds(o, 16)]
                    gs = [plsc.load_gather(bloc, [cc | (d * N)])
                          for d in range(CHUNK)]
                    ps = [g * v for g in gs]
                    for d in range(CHUNK):
                        plsc.addupdate_scatter(accum, [rr | (d * N)], ps[d])
                    return (rn, cn, vn)
                lax.fori_loop(0, PIECE // 16, step, (r0, c0, v0))
            return 0
        lax.fori_loop(0, NP // 2, outer, 0)

        pltpu.async_copy(accum, out_hbm.at[bi, c], semw)
        return 0

    lax.fori_loop(0, TASKS_PER_W, task, 0)
    pltpu.make_async_copy(accum, out_hbm.at[bi, cbase], semw).wait()


@jax.jit
def _spmm(rows, cols, vals, bt):
    mesh = plsc.VectorSubcoreMesh(core_axis_name="c", subcore_axis_name="s",
                                  num_cores=NC, num_subcores=NS)
    f = pl.kernel(
        _body,
        out_type=jax.ShapeDtypeStruct((B, NCHUNK, N * CHUNK), jnp.float32),
        mesh=mesh,
        scratch_types=[
            pltpu.VMEM((N * CHUNK,), jnp.float32),
            pltpu.VMEM((N * CHUNK,), jnp.float32),
            pltpu.VMEM((2 * PIECE + 16,), jnp.int32),
            pltpu.VMEM((2 * PIECE + 16,), jnp.int32),
            pltpu.VMEM((2 * PIECE + 16,), jnp.float32),
            pltpu.SemaphoreType.DMA,
            pltpu.SemaphoreType.DMA,
            pltpu.SemaphoreType.DMA,
        ],
        compiler_params=pltpu.CompilerParams(needs_layout_passes=False),
    )
    return f(rows, cols, vals, bt)


def kernel(indices, values, shape, b):
    nnz = indices.shape[-1]
    rows = indices[:, 0, :].astype(jnp.int32)
    cols = indices[:, 1, :].astype(jnp.int32)
    vals = values.astype(jnp.float32)
    pad = NNZPAD - nnz
    rows = jnp.pad(rows, ((0, 0), (0, pad)))
    cols = jnp.pad(cols, ((0, 0), (0, pad)))
    vals = jnp.pad(vals, ((0, 0), (0, pad)))
    key = (rows & 15) * 16 + ((cols - rows) & 15)
    key = jnp.where(jnp.arange(NNZPAD) < nnz, key, 256)
    order = jnp.argsort(key, axis=-1, stable=True)
    perm = order.reshape(B, 16, NNZPAD // 16).transpose(0, 2, 1) \
                .reshape(B, NNZPAD)
    rows = jnp.take_along_axis(rows, perm, axis=-1)
    cols = jnp.take_along_axis(cols, perm, axis=-1)
    vals = jnp.take_along_axis(vals, perm, axis=-1)
    bt = b.reshape(B, N, NCHUNK, CHUNK).transpose(0, 2, 3, 1) \
         .reshape(B, NCHUNK, CHUNK * N)
    out = _spmm(rows, cols, vals, bt)
    return out.reshape(B, NCHUNK, CHUNK, N).transpose(0, 3, 1, 2) \
              .reshape(B, N, D)
